# Initial kernel scaffold; baseline (speedup 1.0000x reference)
import jax
import jax.numpy as jnp
from jax.experimental import pallas as pl


def kernel(input, Wr, W1_in, b1_in, W2_in, b2_in, W1_out, b1_out, W2_out, b2_out):
    raise NotImplementedError("write your pallas kernel here")


# speedup vs baseline: 1.7876x; 1.7876x over previous
import functools

import jax
import jax.numpy as jnp
from jax import lax
from jax.experimental import pallas as pl
from jax.experimental.pallas import tpu as pltpu
from jax.experimental.pallas import tpu_sc as plsc

T = 2048
D = 1024
E = 8
K = 2
F = 2048
CAP = 640
NR = E * CAP
NSLOT = T * K

NC = 2
NS = 16
NW = NC * NS
L = 16

@functools.lru_cache(maxsize=None)
def _mesh():
    return plsc.VectorSubcoreMesh(core_axis_name="c", subcore_axis_name="s",
                                  num_cores=NC, num_subcores=NS)


def _wid():
    return lax.axis_index("s") * NC + lax.axis_index("c")


def _router_body(x_ref, wr_ref, topi_ref, gates_ref, mp_ref):
    x = x_ref[...]
    wr = wr_ref[...]
    logits = jnp.dot(x, wr, preferred_element_type=jnp.float32)
    m = jnp.max(logits, axis=1, keepdims=True)
    ex = jnp.exp(logits - m)
    probs = ex / jnp.sum(ex, axis=1, keepdims=True)
    iota = lax.broadcasted_iota(jnp.int32, (T, E), 1)
    m1 = jnp.max(probs, axis=1, keepdims=True)
    i1 = jnp.min(jnp.where(probs == m1, iota, E), axis=1, keepdims=True)
    probs2 = jnp.where(iota == i1, -1.0, probs)
    m2 = jnp.max(probs2, axis=1, keepdims=True)
    i2 = jnp.min(jnp.where(probs2 == m2, iota, E), axis=1, keepdims=True)
    s = m1 + m2 + 1e-9
    topi_ref[:, 0:1] = i1
    topi_ref[:, 1:2] = i2
    gates_ref[:, 0:1] = m1 / s
    gates_ref[:, 1:2] = m2 / s
    mp_ref[...] = jnp.sum(probs, axis=0, keepdims=True) * (1.0 / T)


_router = pl.pallas_call(
    _router_body,
    out_shape=[
        jax.ShapeDtypeStruct((T, K), jnp.int32),
        jax.ShapeDtypeStruct((T, K), jnp.float32),
        jax.ShapeDtypeStruct((1, E), jnp.float32),
    ],
)


_CB = 512


def _dispa_body(topi_ref, gates_ref, mp_ref, roww_ref, rowr_ref, aux_ref):
    i1 = topi_ref[:, 0:1]
    i2 = topi_ref[:, 1:2]
    io8 = lax.broadcasted_iota(jnp.int32, (T, E), 1)
    A = (i1 == io8).astype(jnp.float32)
    B = (i2 == io8).astype(jnp.float32)
    Cc = A + B
    cums = []
    for rb in range(T // _CB):
        ri = rb * _CB + lax.broadcasted_iota(jnp.int32, (_CB, T), 0)
        ci = lax.broadcasted_iota(jnp.int32, (_CB, T), 1)
        Lc = (ri > ci).astype(jnp.float32)
        cums.append(jnp.dot(Lc, Cc, preferred_element_type=jnp.float32))
    Cum = jnp.concatenate(cums, axis=0)
    pos0 = jnp.sum(A * Cum, axis=1, keepdims=True)
    pos1 = jnp.sum(B * Cum, axis=1, keepdims=True)
    cnt = jnp.sum(Cc, axis=0, keepdims=True)
    filled = jnp.minimum(cnt, float(CAP))
    mn = jnp.min(filled, axis=1, keepdims=True)
    io18 = lax.broadcasted_iota(jnp.int32, (1, E), 1).astype(jnp.float32)
    esp = jnp.min(jnp.where(filled == mn, io18, float(E)), axis=1,
                  keepdims=True)
    spare = esp * CAP + (CAP - 1)
    row0 = i1.astype(jnp.float32) * CAP + pos0
    row1 = i2.astype(jnp.float32) * CAP + pos1
    v0 = pos0 < CAP
    v1 = pos1 < CAP
    roww_ref[:, 0:1] = jnp.where(v0, row0, -1.0)
    roww_ref[:, 1:2] = jnp.where(v1, row1, -1.0)
    rowr_ref[:, 0:1] = jnp.where(v0, row0, spare).astype(jnp.int32)
    rowr_ref[:, 1:2] = jnp.where(v1, row1, spare).astype(jnp.int32)
    aux_ref[...] = (jnp.sum(mp_ref[...] * cnt, axis=1, keepdims=True)
                    * (float(E) / NSLOT))


_dispa = pl.pallas_call(
    _dispa_body,
    out_shape=[
        jax.ShapeDtypeStruct((T, K), jnp.float32),
        jax.ShapeDtypeStruct((T, K), jnp.int32),
        jax.ShapeDtypeStruct((1, 1), jnp.float32),
    ],
)


_PB = 512


def _dispb_body(rwt_ref, gates_ref, rowrf_ref, out_ref):
    rb = pl.program_id(0)
    pio = rb * _PB + lax.broadcasted_iota(jnp.int32, (_PB, T), 0)
    piof = pio.astype(jnp.float32)
    OH0 = (rwt_ref[0:1, :] == piof).astype(jnp.float32)
    OH1 = (rwt_ref[1:2, :] == piof).astype(jnp.float32)
    tf = lax.broadcasted_iota(jnp.int32, (T, 1), 0).astype(jnp.float32)
    v0 = jnp.concatenate([tf, gates_ref[:, 0:1], rowrf_ref[...]], axis=1)
    v1 = jnp.concatenate([tf, gates_ref[:, 1:2], rowrf_ref[...]], axis=1)
    out_ref[...] = (jnp.dot(OH0, v0, preferred_element_type=jnp.float32)
                    + jnp.dot(OH1, v1, preferred_element_type=jnp.float32))


_dispb = pl.pallas_call(
    _dispb_body,
    grid=(NR // _PB,),
    in_specs=[
        pl.BlockSpec((K, T), lambda rb: (0, 0)),
        pl.BlockSpec((T, K), lambda rb: (0, 0)),
        pl.BlockSpec((T, K), lambda rb: (0, 0)),
    ],
    out_specs=pl.BlockSpec((_PB, 4), lambda rb: (rb, 0)),
    out_shape=jax.ShapeDtypeStruct((NR, 4), jnp.float32),
)


_G_RPT = NR // NW
_G_CH = 32


def _gather_body(src_hbm, idx_hbm, out_hbm, idx_v, idx_c, buf_v, sem):
    wid = _wid()
    base = wid * _G_RPT
    pltpu.sync_copy(idx_hbm.at[pl.ds(base, _G_RPT)], idx_v)
    for c in range(_G_RPT // _G_CH):
        for j in range(_G_CH // L):
            idx_c[pl.ds(j * L, L)] = idx_v[pl.ds(c * _G_CH + j * L, L)]
        pltpu.async_copy(src_hbm.at[idx_c], buf_v, sem).wait()
        pltpu.sync_copy(buf_v, out_hbm.at[pl.ds(base + c * _G_CH, _G_CH)])


@functools.lru_cache(maxsize=None)
def _gather():
    return pl.kernel(
        _gather_body,
        out_type=jax.ShapeDtypeStruct((NR, D), jnp.float32),
        mesh=_mesh(),
        scratch_types=[
            pltpu.VMEM((_G_RPT,), jnp.int32),
            pltpu.VMEM((_G_CH,), jnp.int32),
            pltpu.VMEM((_G_CH, D), jnp.float32),
            pltpu.SemaphoreType.DMA,
        ],
    )


_BF = 512
_NFB = F // _BF


def _ffn_body(xe_ref, w1_ref, b1_ref, w2_ref, b2_ref, g_ref, y_ref):
    fb = pl.program_id(1)
    h = jnp.dot(xe_ref[...], w1_ref[0], preferred_element_type=jnp.float32)
    h = jnp.maximum(h + b1_ref[0], 0.0)
    yp = jnp.dot(h, w2_ref[0], preferred_element_type=jnp.float32)

    @pl.when(fb == 0)
    def _():
        y_ref[...] = yp

    @pl.when(fb != 0)
    def _():
        y_ref[...] = y_ref[...] + yp

    @pl.when(fb == _NFB - 1)
    def _():
        y_ref[...] = (y_ref[...] + b2_ref[0]) * g_ref[0]


_ffn_call = pl.pallas_call(
    _ffn_body,
    grid=(E, _NFB),
    in_specs=[
        pl.BlockSpec((CAP, D), lambda e, fb: (e, 0)),
        pl.BlockSpec((1, D, _BF), lambda e, fb: (e, 0, fb)),
        pl.BlockSpec((1, 1, _BF), lambda e, fb: (e, 0, fb)),
        pl.BlockSpec((1, _BF, D), lambda e, fb: (e, fb, 0)),
        pl.BlockSpec((1, 1, D), lambda e, fb: (e, 0, 0)),
        pl.BlockSpec((1, CAP, 1), lambda e, fb: (e, 0, 0)),
    ],
    out_specs=pl.BlockSpec((CAP, D), lambda e, fb: (e, 0)),
    out_shape=jax.ShapeDtypeStruct((NR, D), jnp.float32),
    compiler_params=pltpu.CompilerParams(
        dimension_semantics=("arbitrary", "arbitrary"),
    ),
)


def _ffn(xe, W1, b1, W2, b2, g3):
    return _ffn_call(xe, W1, b1.reshape(E, 1, F), W2, b2.reshape(E, 1, D), g3)


def _gather2_body(y_hbm, r0_hbm, r1_hbm, out_hbm, i0_v, i1_v, i0_c, i1_c,
                  b_v, sem):
    wid = _wid()
    base = wid * _G_RPT
    pltpu.sync_copy(r0_hbm.at[pl.ds(base, _G_RPT)], i0_v)
    pltpu.sync_copy(r1_hbm.at[pl.ds(base, _G_RPT)], i1_v)
    for c in range(_G_RPT // _G_CH):
        for j in range(_G_CH // L):
            off = c * _G_CH + j * L
            i0_c[pl.ds(j * L, L)] = i0_v[pl.ds(off, L)]
            i1_c[pl.ds(j * L, L)] = i1_v[pl.ds(off, L)]
        pltpu.async_copy(y_hbm.at[i0_c], b_v, sem).wait()
        pltpu.async_copy(y_hbm.at[i1_c], b_v, sem, add=True).wait()
        pltpu.sync_copy(b_v, out_hbm.at[pl.ds(base + c * _G_CH, _G_CH)])


@functools.lru_cache(maxsize=None)
def _gather2():
    return pl.kernel(
        _gather2_body,
        out_type=jax.ShapeDtypeStruct((NR, D), jnp.float32),
        mesh=_mesh(),
        scratch_types=[
            pltpu.VMEM((_G_RPT,), jnp.int32),
            pltpu.VMEM((_G_RPT,), jnp.int32),
            pltpu.VMEM((_G_CH,), jnp.int32),
            pltpu.VMEM((_G_CH,), jnp.int32),
            pltpu.VMEM((_G_CH, D), jnp.float32),
            pltpu.SemaphoreType.DMA,
        ],
    )


_C_TPT = T // NW


def _combine_body(y_hbm, r0_hbm, r1_hbm, out_hbm, i0_v, i1_v, b_v, sem):
    wid = _wid()
    tbase = wid * _C_TPT
    pltpu.sync_copy(r0_hbm.at[pl.ds(tbase, _C_TPT)], i0_v)
    pltpu.sync_copy(r1_hbm.at[pl.ds(tbase, _C_TPT)], i1_v)
    pltpu.async_copy(y_hbm.at[i0_v], b_v, sem).wait()
    pltpu.async_copy(y_hbm.at[i1_v], b_v, sem, add=True).wait()
    pltpu.sync_copy(b_v, out_hbm.at[pl.ds(tbase, _C_TPT)])


@functools.lru_cache(maxsize=None)
def _combine():
    return pl.kernel(
        _combine_body,
        out_type=jax.ShapeDtypeStruct((T, D), jnp.float32),
        mesh=_mesh(),
        scratch_types=[
            pltpu.VMEM((_C_TPT,), jnp.int32),
            pltpu.VMEM((_C_TPT,), jnp.int32),
            pltpu.VMEM((_C_TPT, D), jnp.float32),
            pltpu.SemaphoreType.DMA,
        ],
    )


def kernel(input, Wr, W1_in, b1_in, W2_in, b2_in, W1_out, b1_out, W2_out,
           b2_out):
    topi, gates, mp = _router(input, Wr)
    roww, rowr, aux11 = _dispa(topi, gates, mp)
    pk = _dispb(roww.T, gates, rowr.astype(jnp.float32))
    tok_idx = pk[:, 0].astype(jnp.int32)
    g3 = pk[:, 1].reshape(E, CAP, 1)
    rr0 = pk[:, 2].astype(jnp.int32)
    rr1 = pk[:, 3].astype(jnp.int32)
    row0 = rowr[:, 0]
    row1 = rowr[:, 1]
    xe = _gather()(input, tok_idx)
    y1 = _ffn(xe, W1_in, b1_in, W2_in, b2_in, g3)
    xe2 = _gather2()(y1, rr0, rr1)
    y2 = _ffn(xe2, W1_out, b1_out, W2_out, b2_out, g3)
    output = _combine()(y2, row0, row1)
    return output, aux11[0, 0]


# speedup vs baseline: 1.7931x; 1.0031x over previous
import functools

import jax
import jax.numpy as jnp
from jax import lax
from jax.experimental import pallas as pl
from jax.experimental.pallas import tpu as pltpu
from jax.experimental.pallas import tpu_sc as plsc

T = 2048
D = 1024
E = 8
K = 2
F = 2048
CAP = 640
NR = E * CAP
NSLOT = T * K

NC = 2
NS = 16
NW = NC * NS
L = 16

@functools.lru_cache(maxsize=None)
def _mesh():
    return plsc.VectorSubcoreMesh(core_axis_name="c", subcore_axis_name="s",
                                  num_cores=NC, num_subcores=NS)


def _wid():
    return lax.axis_index("s") * NC + lax.axis_index("c")


def _router_body(x_ref, wr_ref, topi_ref, gates_ref, mp_ref):
    x = x_ref[...]
    wr = wr_ref[...]
    logits = jnp.dot(x, wr, preferred_element_type=jnp.float32)
    m = jnp.max(logits, axis=1, keepdims=True)
    ex = jnp.exp(logits - m)
    probs = ex / jnp.sum(ex, axis=1, keepdims=True)
    iota = lax.broadcasted_iota(jnp.int32, (T, E), 1)
    m1 = jnp.max(probs, axis=1, keepdims=True)
    i1 = jnp.min(jnp.where(probs == m1, iota, E), axis=1, keepdims=True)
    probs2 = jnp.where(iota == i1, -1.0, probs)
    m2 = jnp.max(probs2, axis=1, keepdims=True)
    i2 = jnp.min(jnp.where(probs2 == m2, iota, E), axis=1, keepdims=True)
    s = m1 + m2 + 1e-9
    topi_ref[:, 0:1] = i1
    topi_ref[:, 1:2] = i2
    gates_ref[:, 0:1] = m1 / s
    gates_ref[:, 1:2] = m2 / s
    mp_ref[...] = jnp.sum(probs, axis=0, keepdims=True) * (1.0 / T)


_router = pl.pallas_call(
    _router_body,
    out_shape=[
        jax.ShapeDtypeStruct((T, K), jnp.int32),
        jax.ShapeDtypeStruct((T, K), jnp.float32),
        jax.ShapeDtypeStruct((1, E), jnp.float32),
    ],
)


_CB = 512


def _dispa_body(topi_ref, gates_ref, mp_ref, roww_ref, rowr_ref, aux_ref):
    i1 = topi_ref[:, 0:1]
    i2 = topi_ref[:, 1:2]
    io8 = lax.broadcasted_iota(jnp.int32, (T, E), 1)
    A = (i1 == io8).astype(jnp.float32)
    B = (i2 == io8).astype(jnp.float32)
    Cc = A + B
    cums = []
    for rb in range(T // _CB):
        ri = rb * _CB + lax.broadcasted_iota(jnp.int32, (_CB, T), 0)
        ci = lax.broadcasted_iota(jnp.int32, (_CB, T), 1)
        Lc = (ri > ci).astype(jnp.float32)
        cums.append(jnp.dot(Lc, Cc, preferred_element_type=jnp.float32))
    Cum = jnp.concatenate(cums, axis=0)
    pos0 = jnp.sum(A * Cum, axis=1, keepdims=True)
    pos1 = jnp.sum(B * Cum, axis=1, keepdims=True)
    cnt = jnp.sum(Cc, axis=0, keepdims=True)
    filled = jnp.minimum(cnt, float(CAP))
    mn = jnp.min(filled, axis=1, keepdims=True)
    io18 = lax.broadcasted_iota(jnp.int32, (1, E), 1).astype(jnp.float32)
    esp = jnp.min(jnp.where(filled == mn, io18, float(E)), axis=1,
                  keepdims=True)
    spare = esp * CAP + (CAP - 1)
    row0 = i1.astype(jnp.float32) * CAP + pos0
    row1 = i2.astype(jnp.float32) * CAP + pos1
    v0 = pos0 < CAP
    v1 = pos1 < CAP
    roww_ref[:, 0:1] = jnp.where(v0, row0, -1.0)
    roww_ref[:, 1:2] = jnp.where(v1, row1, -1.0)
    rowr_ref[:, 0:1] = jnp.where(v0, row0, spare).astype(jnp.int32)
    rowr_ref[:, 1:2] = jnp.where(v1, row1, spare).astype(jnp.int32)
    aux_ref[...] = (jnp.sum(mp_ref[...] * cnt, axis=1, keepdims=True)
                    * (float(E) / NSLOT))


_dispa = pl.pallas_call(
    _dispa_body,
    out_shape=[
        jax.ShapeDtypeStruct((T, K), jnp.float32),
        jax.ShapeDtypeStruct((T, K), jnp.int32),
        jax.ShapeDtypeStruct((1, 1), jnp.float32),
    ],
)


_PB = 512


def _dispb_body(rwt_ref, gates_ref, rowrf_ref, out_ref):
    rb = pl.program_id(0)
    pio = rb * _PB + lax.broadcasted_iota(jnp.int32, (_PB, T), 0)
    piof = pio.astype(jnp.float32)
    OH0 = (rwt_ref[0:1, :] == piof).astype(jnp.float32)
    OH1 = (rwt_ref[1:2, :] == piof).astype(jnp.float32)
    tf = lax.broadcasted_iota(jnp.int32, (T, 1), 0).astype(jnp.float32)
    v0 = jnp.concatenate([tf, gates_ref[:, 0:1], rowrf_ref[...]], axis=1)
    v1 = jnp.concatenate([tf, gates_ref[:, 1:2], rowrf_ref[...]], axis=1)
    out_ref[...] = (jnp.dot(OH0, v0, preferred_element_type=jnp.float32)
                    + jnp.dot(OH1, v1, preferred_element_type=jnp.float32))


_dispb = pl.pallas_call(
    _dispb_body,
    grid=(NR // _PB,),
    in_specs=[
        pl.BlockSpec((K, T), lambda rb: (0, 0)),
        pl.BlockSpec((T, K), lambda rb: (0, 0)),
        pl.BlockSpec((T, K), lambda rb: (0, 0)),
    ],
    out_specs=pl.BlockSpec((_PB, 4), lambda rb: (rb, 0)),
    out_shape=jax.ShapeDtypeStruct((NR, 4), jnp.float32),
)


_G_RPT = NR // NW
_G_CH = 80


def _gather_body(src_hbm, idx_hbm, out_hbm, idx_v, idx_c, buf_v, sem):
    wid = _wid()
    base = wid * _G_RPT
    pltpu.sync_copy(idx_hbm.at[pl.ds(base, _G_RPT)], idx_v)
    for c in range(_G_RPT // _G_CH):
        for j in range(_G_CH // L):
            idx_c[pl.ds(j * L, L)] = idx_v[pl.ds(c * _G_CH + j * L, L)]
        pltpu.async_copy(src_hbm.at[idx_c], buf_v, sem).wait()
        pltpu.sync_copy(buf_v, out_hbm.at[pl.ds(base + c * _G_CH, _G_CH)])


@functools.lru_cache(maxsize=None)
def _gather():
    return pl.kernel(
        _gather_body,
        out_type=jax.ShapeDtypeStruct((NR, D), jnp.float32),
        mesh=_mesh(),
        scratch_types=[
            pltpu.VMEM((_G_RPT,), jnp.int32),
            pltpu.VMEM((_G_CH,), jnp.int32),
            pltpu.VMEM((_G_CH, D), jnp.float32),
            pltpu.SemaphoreType.DMA,
        ],
    )


_BF = 512
_NFB = F // _BF


def _ffn_body(xe_ref, w1_ref, b1_ref, w2_ref, b2_ref, g_ref, y_ref):
    fb = pl.program_id(1)
    h = jnp.dot(xe_ref[...], w1_ref[0], preferred_element_type=jnp.float32)
    h = jnp.maximum(h + b1_ref[0], 0.0)
    yp = jnp.dot(h, w2_ref[0], preferred_element_type=jnp.float32)

    @pl.when(fb == 0)
    def _():
        y_ref[...] = yp

    @pl.when(fb != 0)
    def _():
        y_ref[...] = y_ref[...] + yp

    @pl.when(fb == _NFB - 1)
    def _():
        y_ref[...] = (y_ref[...] + b2_ref[0]) * g_ref[0]


_ffn_call = pl.pallas_call(
    _ffn_body,
    grid=(E, _NFB),
    in_specs=[
        pl.BlockSpec((CAP, D), lambda e, fb: (e, 0)),
        pl.BlockSpec((1, D, _BF), lambda e, fb: (e, 0, fb)),
        pl.BlockSpec((1, 1, _BF), lambda e, fb: (e, 0, fb)),
        pl.BlockSpec((1, _BF, D), lambda e, fb: (e, fb, 0)),
        pl.BlockSpec((1, 1, D), lambda e, fb: (e, 0, 0)),
        pl.BlockSpec((1, CAP, 1), lambda e, fb: (e, 0, 0)),
    ],
    out_specs=pl.BlockSpec((CAP, D), lambda e, fb: (e, 0)),
    out_shape=jax.ShapeDtypeStruct((NR, D), jnp.float32),
    compiler_params=pltpu.CompilerParams(
        dimension_semantics=("arbitrary", "arbitrary"),
    ),
)


def _ffn(xe, W1, b1, W2, b2, g3):
    return _ffn_call(xe, W1, b1.reshape(E, 1, F), W2, b2.reshape(E, 1, D), g3)


def _gather2_body(y_hbm, r0_hbm, r1_hbm, out_hbm, i0_v, i1_v, i0_c, i1_c,
                  b_v, sem):
    wid = _wid()
    base = wid * _G_RPT
    pltpu.sync_copy(r0_hbm.at[pl.ds(base, _G_RPT)], i0_v)
    pltpu.sync_copy(r1_hbm.at[pl.ds(base, _G_RPT)], i1_v)
    for c in range(_G_RPT // _G_CH):
        for j in range(_G_CH // L):
            off = c * _G_CH + j * L
            i0_c[pl.ds(j * L, L)] = i0_v[pl.ds(off, L)]
            i1_c[pl.ds(j * L, L)] = i1_v[pl.ds(off, L)]
        pltpu.async_copy(y_hbm.at[i0_c], b_v, sem).wait()
        pltpu.async_copy(y_hbm.at[i1_c], b_v, sem, add=True).wait()
        pltpu.sync_copy(b_v, out_hbm.at[pl.ds(base + c * _G_CH, _G_CH)])


@functools.lru_cache(maxsize=None)
def _gather2():
    return pl.kernel(
        _gather2_body,
        out_type=jax.ShapeDtypeStruct((NR, D), jnp.float32),
        mesh=_mesh(),
        scratch_types=[
            pltpu.VMEM((_G_RPT,), jnp.int32),
            pltpu.VMEM((_G_RPT,), jnp.int32),
            pltpu.VMEM((_G_CH,), jnp.int32),
            pltpu.VMEM((_G_CH,), jnp.int32),
            pltpu.VMEM((_G_CH, D), jnp.float32),
            pltpu.SemaphoreType.DMA,
        ],
    )


_C_TPT = T // NW


def _combine_body(y_hbm, r0_hbm, r1_hbm, out_hbm, i0_v, i1_v, b_v, sem):
    wid = _wid()
    tbase = wid * _C_TPT
    pltpu.sync_copy(r0_hbm.at[pl.ds(tbase, _C_TPT)], i0_v)
    pltpu.sync_copy(r1_hbm.at[pl.ds(tbase, _C_TPT)], i1_v)
    pltpu.async_copy(y_hbm.at[i0_v], b_v, sem).wait()
    pltpu.async_copy(y_hbm.at[i1_v], b_v, sem, add=True).wait()
    pltpu.sync_copy(b_v, out_hbm.at[pl.ds(tbase, _C_TPT)])


@functools.lru_cache(maxsize=None)
def _combine():
    return pl.kernel(
        _combine_body,
        out_type=jax.ShapeDtypeStruct((T, D), jnp.float32),
        mesh=_mesh(),
        scratch_types=[
            pltpu.VMEM((_C_TPT,), jnp.int32),
            pltpu.VMEM((_C_TPT,), jnp.int32),
            pltpu.VMEM((_C_TPT, D), jnp.float32),
            pltpu.SemaphoreType.DMA,
        ],
    )


def kernel(input, Wr, W1_in, b1_in, W2_in, b2_in, W1_out, b1_out, W2_out,
           b2_out):
    topi, gates, mp = _router(input, Wr)
    roww, rowr, aux11 = _dispa(topi, gates, mp)
    pk = _dispb(roww.T, gates, rowr.astype(jnp.float32))
    tok_idx = pk[:, 0].astype(jnp.int32)
    g3 = pk[:, 1].reshape(E, CAP, 1)
    rr0 = pk[:, 2].astype(jnp.int32)
    rr1 = pk[:, 3].astype(jnp.int32)
    row0 = rowr[:, 0]
    row1 = rowr[:, 1]
    xe = _gather()(input, tok_idx)
    y1 = _ffn(xe, W1_in, b1_in, W2_in, b2_in, g3)
    xe2 = _gather2()(y1, rr0, rr1)
    y2 = _ffn(xe2, W1_out, b1_out, W2_out, b2_out, g3)
    output = _combine()(y2, row0, row1)
    return output, aux11[0, 0]
